# Initial kernel scaffold; baseline (speedup 1.0000x reference)
import jax
import jax.numpy as jnp
from jax.experimental import pallas as pl


def kernel(word_id_seq, W_word, char_emb_tensor):
    raise NotImplementedError("write your pallas kernel here")


# speedup vs baseline: 1.2880x; 1.2880x over previous
import functools

import jax
import jax.numpy as jnp
from jax import lax
from jax.experimental import pallas as pl
from jax.experimental.pallas import tpu as pltpu
from jax.experimental.pallas import tpu_sc as plsc

VOCAB = 100000
WORD_DIM = 300
CHAR_DIM = 100
EXT_DIM = WORD_DIM + CHAR_DIM
BATCH = 1024
MAX_SEQ = 200
NTOK = BATCH * MAX_SEQ

_ROWS_PER_BLOCK = 1000


def _concat_body(w_ref, c_ref, o_ref):
    o_ref[...] = jnp.concatenate([w_ref[...], c_ref[...]], axis=-1)


def _build_extended(W_word, char_emb_tensor):
    grid = (VOCAB // _ROWS_PER_BLOCK,)
    return pl.pallas_call(
        _concat_body,
        grid=grid,
        in_specs=[
            pl.BlockSpec((_ROWS_PER_BLOCK, WORD_DIM), lambda i: (i, 0)),
            pl.BlockSpec((_ROWS_PER_BLOCK, CHAR_DIM), lambda i: (i, 0)),
        ],
        out_specs=pl.BlockSpec((_ROWS_PER_BLOCK, EXT_DIM), lambda i: (i, 0)),
        out_shape=jax.ShapeDtypeStruct((VOCAB, EXT_DIM), jnp.float32),
    )(W_word, char_emb_tensor)


_NW = 32
_PER_W = NTOK // _NW
_CH = 128
_NCHUNK = _PER_W // _CH


def _make_sc_gather():
    mesh = plsc.VectorSubcoreMesh(core_axis_name="c", subcore_axis_name="s")

    @functools.partial(
        pl.kernel,
        mesh=mesh,
        compiler_params=pltpu.CompilerParams(use_tc_tiling_on_sc=False),
        out_type=jax.ShapeDtypeStruct((NTOK, EXT_DIM), jnp.float32),
        scratch_types=[
            pltpu.VMEM((_CH,), jnp.int32),
            pltpu.VMEM((_CH, EXT_DIM), jnp.float32),
            pltpu.SemaphoreType.DMA,
        ],
    )
    def gather_k(table_hbm, idx_hbm, out_hbm, idx_v, rows_v, sem):
        wid = lax.axis_index("s") * 2 + lax.axis_index("c")
        base = wid * _PER_W

        def body(i, carry):
            off = base + i * _CH
            pltpu.sync_copy(idx_hbm.at[pl.ds(off, _CH)], idx_v)
            pltpu.async_copy(table_hbm.at[idx_v], rows_v, sem).wait()
            pltpu.sync_copy(rows_v, out_hbm.at[pl.ds(off, _CH)])
            return carry

        lax.fori_loop(0, _NCHUNK, body, 0)

    return gather_k


_sc_gather = _make_sc_gather()


def kernel(word_id_seq, W_word, char_emb_tensor):
    ext = _build_extended(W_word, char_emb_tensor)
    flat_ids = word_id_seq.reshape(NTOK).astype(jnp.int32)
    out = _sc_gather(ext, flat_ids)
    embedded_seq = out.reshape(BATCH, MAX_SEQ, EXT_DIM)
    return (embedded_seq, ext, word_id_seq)


# speedup vs baseline: 1.3220x; 1.0265x over previous
import functools

import jax
import jax.numpy as jnp
from jax import lax
from jax.experimental import pallas as pl
from jax.experimental.pallas import tpu as pltpu
from jax.experimental.pallas import tpu_sc as plsc

VOCAB = 100000
WORD_DIM = 300
CHAR_DIM = 100
EXT_DIM = WORD_DIM + CHAR_DIM
BATCH = 1024
MAX_SEQ = 200
NTOK = BATCH * MAX_SEQ

_ROWS_PER_BLOCK = 1000


def _concat_body(w_ref, c_ref, o_ref):
    o_ref[...] = jnp.concatenate([w_ref[...], c_ref[...]], axis=-1)


def _build_extended(W_word, char_emb_tensor):
    grid = (VOCAB // _ROWS_PER_BLOCK,)
    return pl.pallas_call(
        _concat_body,
        grid=grid,
        in_specs=[
            pl.BlockSpec((_ROWS_PER_BLOCK, WORD_DIM), lambda i: (i, 0)),
            pl.BlockSpec((_ROWS_PER_BLOCK, CHAR_DIM), lambda i: (i, 0)),
        ],
        out_specs=pl.BlockSpec((_ROWS_PER_BLOCK, EXT_DIM), lambda i: (i, 0)),
        out_shape=jax.ShapeDtypeStruct((VOCAB, EXT_DIM), jnp.float32),
    )(W_word, char_emb_tensor)


_NW = 32
_PER_W = NTOK // _NW
_CH = 128
_NCHUNK = _PER_W // _CH


def _make_sc_gather():
    mesh = plsc.VectorSubcoreMesh(core_axis_name="c", subcore_axis_name="s")

    @functools.partial(
        pl.kernel,
        mesh=mesh,
        compiler_params=pltpu.CompilerParams(use_tc_tiling_on_sc=False),
        out_type=jax.ShapeDtypeStruct((NTOK, EXT_DIM), jnp.float32),
        scratch_types=[
            pltpu.VMEM((_PER_W,), jnp.int32),
            pltpu.VMEM((_CH, EXT_DIM), jnp.float32),
            pltpu.VMEM((_CH, EXT_DIM), jnp.float32),
            pltpu.SemaphoreType.DMA,
            pltpu.SemaphoreType.DMA,
        ],
    )
    def gather_k(table_hbm, idx_hbm, out_hbm, idx_v, rows0, rows1, sem0, sem1):
        wid = lax.axis_index("s") * 2 + lax.axis_index("c")
        base = wid * _PER_W
        rows = (rows0, rows1)
        sems = (sem0, sem1)

        pltpu.sync_copy(idx_hbm.at[pl.ds(base, _PER_W)], idx_v)

        def fire(i, b):
            idx_sl = idx_v.at[pl.ds(i * _CH, _CH)]
            pltpu.async_copy(table_hbm.at[idx_sl], rows[b], sems[b])

        def drain_and_scatter(i, b):
            pltpu.make_async_copy(table_hbm.at[pl.ds(0, _CH)], rows[b], sems[b]).wait()
            pltpu.sync_copy(rows[b], out_hbm.at[pl.ds(base + i * _CH, _CH)])

        for b in range(2):
            fire(b, b)

        def body(i0, carry):
            for b in range(2):
                i = i0 + b
                drain_and_scatter(i, b)
                fire(i + 2, b)
            return carry

        lax.fori_loop(0, (_NCHUNK - 2) // 2, lambda j, c: body(j * 2, c), 0)
        for b in range(2):
            drain_and_scatter(_NCHUNK - 2 + b, b)

    return gather_k


_sc_gather = _make_sc_gather()


def kernel(word_id_seq, W_word, char_emb_tensor):
    ext = _build_extended(W_word, char_emb_tensor)
    flat_ids = word_id_seq.reshape(NTOK).astype(jnp.int32)
    out = _sc_gather(ext, flat_ids)
    embedded_seq = out.reshape(BATCH, MAX_SEQ, EXT_DIM)
    return (embedded_seq, ext, word_id_seq)
